# Initial kernel scaffold; baseline (speedup 1.0000x reference)
import jax
import jax.numpy as jnp
from jax.experimental import pallas as pl


def kernel(w_cat, b_cat, g_seq, hidden0):
    raise NotImplementedError("write your pallas kernel here")


# speedup vs baseline: 2.0149x; 2.0149x over previous
import functools

import jax
import jax.numpy as jnp
from jax.experimental import pallas as pl
from jax.experimental.pallas import tpu as pltpu

_LANE = 128
_SUB = 8


def _ceil_to(x, m):
    return ((x + m - 1) // m) * m


def _block_body(g_ref, h0_ref, w_ref, b_ref, out_ref, x_ref, *, gk, nt):
    @pl.when(pl.program_id(0) == 0)
    def _seed():
        x_ref[:, gk:] = h0_ref[...]

    for s in range(nt):
        x_ref[:, :gk] = g_ref[s]
        h = jnp.maximum(
            jnp.dot(x_ref[...], w_ref[...],
                    preferred_element_type=jnp.float32) + b_ref[...],
            0.0,
        )
        x_ref[:, gk:] = h
        out_ref[s] = h.astype(out_ref.dtype)


def kernel(w_cat, b_cat, g_seq, hidden0):
    T, B, input_size = g_seq.shape
    hidden_size = hidden0.shape[1]
    h_pad = w_cat.shape[1]
    in_pad = w_cat.shape[0] - h_pad
    k_pad = in_pad + h_pad
    b_pad = _ceil_to(B, _SUB)

    nt = 1
    while nt < 8 and T % (nt * 2) == 0:
        nt *= 2

    g_p = g_seq.astype(jnp.float32)
    if (b_pad, in_pad) != (B, input_size):
        g_p = jnp.zeros((T, b_pad, in_pad), jnp.float32).at[
            :, :B, :input_size].set(g_p)
    h0_p = hidden0.astype(jnp.float32)
    if (b_pad, h_pad) != (B, hidden_size):
        h0_p = jnp.zeros((b_pad, h_pad), jnp.float32).at[
            :B, :hidden_size].set(h0_p)

    body = functools.partial(_block_body, gk=in_pad, nt=nt)

    h_seq = pl.pallas_call(
        body,
        out_shape=jax.ShapeDtypeStruct((T, b_pad, h_pad), jnp.float32),
        grid=(T // nt,),
        in_specs=[
            pl.BlockSpec((nt, b_pad, in_pad), lambda i: (i, 0, 0)),
            pl.BlockSpec((b_pad, h_pad), lambda i: (0, 0)),
            pl.BlockSpec((k_pad, h_pad), lambda i: (0, 0)),
            pl.BlockSpec((1, h_pad), lambda i: (0, 0)),
        ],
        out_specs=pl.BlockSpec((nt, b_pad, h_pad), lambda i: (i, 0, 0)),
        scratch_shapes=[pltpu.VMEM((b_pad, k_pad), jnp.float32)],
        compiler_params=pltpu.CompilerParams(
            dimension_semantics=("arbitrary",)),
    )(g_p, h0_p, w_cat.astype(jnp.float32), b_cat.astype(jnp.float32))

    if (b_pad, h_pad) != (B, hidden_size):
        h_seq = h_seq[:, :B, :hidden_size]
    return h_seq


# speedup vs baseline: 2.0669x; 1.0258x over previous
import functools

import jax
import jax.numpy as jnp
from jax.experimental import pallas as pl
from jax.experimental.pallas import tpu as pltpu

_LANE = 128
_SUB = 8


def _ceil_to(x, m):
    return ((x + m - 1) // m) * m


def _block_body(g_ref, h0_ref, w_ref, b_ref, out_ref, x_ref, *, gk, nt):
    @pl.when(pl.program_id(0) == 0)
    def _seed():
        x_ref[:, gk:] = h0_ref[...]

    for s in range(nt):
        x_ref[:, :gk] = g_ref[s]
        h = jnp.maximum(
            jnp.dot(x_ref[...], w_ref[...],
                    preferred_element_type=jnp.float32) + b_ref[...],
            0.0,
        )
        x_ref[:, gk:] = h
        out_ref[s] = h.astype(out_ref.dtype)


def kernel(w_cat, b_cat, g_seq, hidden0):
    T, B, input_size = g_seq.shape
    hidden_size = hidden0.shape[1]
    h_pad = w_cat.shape[1]
    in_pad = w_cat.shape[0] - h_pad
    k_pad = in_pad + h_pad
    b_pad = _ceil_to(B, _SUB)

    nt = 1
    while nt < 16 and T % (nt * 2) == 0:
        nt *= 2

    g_p = g_seq.astype(jnp.float32)
    if (b_pad, in_pad) != (B, input_size):
        g_p = jnp.zeros((T, b_pad, in_pad), jnp.float32).at[
            :, :B, :input_size].set(g_p)
    h0_p = hidden0.astype(jnp.float32)
    if (b_pad, h_pad) != (B, hidden_size):
        h0_p = jnp.zeros((b_pad, h_pad), jnp.float32).at[
            :B, :hidden_size].set(h0_p)

    body = functools.partial(_block_body, gk=in_pad, nt=nt)

    h_seq = pl.pallas_call(
        body,
        out_shape=jax.ShapeDtypeStruct((T, b_pad, h_pad), jnp.float32),
        grid=(T // nt,),
        in_specs=[
            pl.BlockSpec((nt, b_pad, in_pad), lambda i: (i, 0, 0)),
            pl.BlockSpec((b_pad, h_pad), lambda i: (0, 0)),
            pl.BlockSpec((k_pad, h_pad), lambda i: (0, 0)),
            pl.BlockSpec((1, h_pad), lambda i: (0, 0)),
        ],
        out_specs=pl.BlockSpec((nt, b_pad, h_pad), lambda i: (i, 0, 0)),
        scratch_shapes=[pltpu.VMEM((b_pad, k_pad), jnp.float32)],
        compiler_params=pltpu.CompilerParams(
            dimension_semantics=("arbitrary",)),
    )(g_p, h0_p, w_cat.astype(jnp.float32), b_cat.astype(jnp.float32))

    if (b_pad, h_pad) != (B, hidden_size):
        h_seq = h_seq[:, :B, :hidden_size]
    return h_seq
